# baseline (speedup 1.0000x reference)
import jax
import jax.numpy as jnp
from jax import lax
from jax.experimental import pallas as pl
from jax.experimental.pallas import tpu as pltpu
from jax.experimental.pallas import tpu_sc as plsc

BATCH = 16384
D = 128
NC = 2
NS = 16
NW = NC * NS
BPW = BATCH // NW
C = 128
NCHUNK = BPW // C
NBUF = 2
LANES = 16


def _gmf_body(uid_hbm, iid_hbm, ut_hbm, it_hbm, out_hbm,
              idx_u, idx_i, u_rows, i_rows, o_rows,
              sg0, sg1, so0, so1, sem_idx):
    sem_g = [sg0, sg1]
    sem_o = [so0, so1]
    wid = lax.axis_index("s") * NC + lax.axis_index("c")
    base = wid * BPW

    ci_u = pltpu.async_copy(uid_hbm.at[pl.ds(base, BPW)], idx_u, sem_idx)
    ci_i = pltpu.async_copy(iid_hbm.at[pl.ds(base, BPW)], idx_i, sem_idx)
    ci_u.wait()
    ci_i.wait()

    def issue_gather(k):
        b = k % NBUF
        sl = pl.ds(k * C, C)
        cu = pltpu.async_copy(ut_hbm.at[idx_u.at[sl]], u_rows.at[b], sem_g[b])
        ci = pltpu.async_copy(it_hbm.at[idx_i.at[sl]], i_rows.at[b], sem_g[b])
        return (cu, ci)

    pend_g = [None] * NCHUNK
    pend_o = [None] * NCHUNK
    pend_g[0] = issue_gather(0)
    pend_g[1] = issue_gather(1)
    for k in range(NCHUNK):
        b = k % NBUF
        cu, ci = pend_g[k]
        cu.wait()
        ci.wait()
        if k >= NBUF:
            pa, pb = pend_o[k - NBUF]
            pa.wait()
            pb.wait()

        H = C // 2

        @plsc.parallel_loop(0, H, step=1, unroll=4)
        def _(r):
            for c in range(D // LANES):
                sl = pl.ds(c * LANES, LANES)
                o_rows[b, r, sl] = u_rows[b, r, sl] * i_rows[b, r, sl]

        off = base + k * C
        po_a = pltpu.async_copy(
            o_rows.at[b, pl.ds(0, H)], out_hbm.at[pl.ds(off, H)], sem_o[b])

        @plsc.parallel_loop(H, C, step=1, unroll=4)
        def _(r):
            for c in range(D // LANES):
                sl = pl.ds(c * LANES, LANES)
                o_rows[b, r, sl] = u_rows[b, r, sl] * i_rows[b, r, sl]

        if k + NBUF < NCHUNK:
            pend_g[k + NBUF] = issue_gather(k + NBUF)
        po_b = pltpu.async_copy(
            o_rows.at[b, pl.ds(H, H)], out_hbm.at[pl.ds(off + H, H)], sem_o[b])
        pend_o[k] = (po_a, po_b)
    for k in range(max(0, NCHUNK - NBUF), NCHUNK):
        pa, pb = pend_o[k]
        pa.wait()
        pb.wait()


def kernel(user_ids, item_ids, user_table, item_table):
    mesh = plsc.VectorSubcoreMesh(core_axis_name="c", subcore_axis_name="s")
    f = pl.kernel(
        _gmf_body,
        out_type=jax.ShapeDtypeStruct((BATCH, D), jnp.float32),
        mesh=mesh,
        scratch_types=[
            pltpu.VMEM((BPW,), jnp.int32),
            pltpu.VMEM((BPW,), jnp.int32),
            pltpu.VMEM((NBUF, C, D), jnp.float32),
            pltpu.VMEM((NBUF, C, D), jnp.float32),
            pltpu.VMEM((NBUF, C, D), jnp.float32),
            pltpu.SemaphoreType.DMA,
            pltpu.SemaphoreType.DMA,
            pltpu.SemaphoreType.DMA,
            pltpu.SemaphoreType.DMA,
            pltpu.SemaphoreType.DMA,
        ],
    )
    return f(user_ids.astype(jnp.int32), item_ids.astype(jnp.int32),
             user_table, item_table)


# speedup vs baseline: 1.0305x; 1.0305x over previous
import jax
import jax.numpy as jnp
from jax import lax
from jax.experimental import pallas as pl
from jax.experimental.pallas import tpu as pltpu
from jax.experimental.pallas import tpu_sc as plsc

BATCH = 16384
D = 128
NC = 2
NS = 16
NW = NC * NS
BPW = BATCH // NW
C = 128
NCHUNK = BPW // C
NBUF = 2
LANES = 16


def _gmf_body(uid_hbm, iid_hbm, ut_hbm, it_hbm, out_hbm,
              idx_u, idx_i, u_rows, i_rows, o_rows,
              sg0, sg1, so0, so1, sem_idx):
    sem_g = [sg0, sg1]
    sem_o = [so0, so1]
    wid = lax.axis_index("s") * NC + lax.axis_index("c")
    base = wid * BPW

    ci_u = pltpu.async_copy(uid_hbm.at[pl.ds(base, BPW)], idx_u, sem_idx)
    ci_i = pltpu.async_copy(iid_hbm.at[pl.ds(base, BPW)], idx_i, sem_idx)
    ci_u.wait()
    ci_i.wait()

    def issue_gather(k):
        b = k % NBUF
        sl = pl.ds(k * C, C)
        cu = pltpu.async_copy(ut_hbm.at[idx_u.at[sl]], u_rows.at[b], sem_g[b])
        ci = pltpu.async_copy(it_hbm.at[idx_i.at[sl]], i_rows.at[b], sem_g[b])
        return (cu, ci)

    pend_g = [None] * NCHUNK
    pend_o = [None] * NCHUNK
    pend_g[0] = issue_gather(0)
    pend_g[1] = issue_gather(1)
    for k in range(NCHUNK):
        b = k % NBUF
        cu, ci = pend_g[k]
        cu.wait()
        ci.wait()
        if k >= NBUF:
            pend_o[k - NBUF].wait()

        @plsc.parallel_loop(0, C, step=1, unroll=4)
        def _(r):
            for c in range(D // LANES):
                sl = pl.ds(c * LANES, LANES)
                o_rows[b, r, sl] = u_rows[b, r, sl] * i_rows[b, r, sl]

        if k + NBUF < NCHUNK:
            pend_g[k + NBUF] = issue_gather(k + NBUF)
        off = base + k * C
        pend_o[k] = pltpu.async_copy(
            o_rows.at[b], out_hbm.at[pl.ds(off, C)], sem_o[b])
    for k in range(max(0, NCHUNK - NBUF), NCHUNK):
        pend_o[k].wait()


def kernel(user_ids, item_ids, user_table, item_table):
    mesh = plsc.VectorSubcoreMesh(core_axis_name="c", subcore_axis_name="s")
    f = pl.kernel(
        _gmf_body,
        out_type=jax.ShapeDtypeStruct((BATCH, D), jnp.float32),
        mesh=mesh,
        scratch_types=[
            pltpu.VMEM((BPW,), jnp.int32),
            pltpu.VMEM((BPW,), jnp.int32),
            pltpu.VMEM((NBUF, C, D), jnp.float32),
            pltpu.VMEM((NBUF, C, D), jnp.float32),
            pltpu.VMEM((NBUF, C, D), jnp.float32),
            pltpu.SemaphoreType.DMA,
            pltpu.SemaphoreType.DMA,
            pltpu.SemaphoreType.DMA,
            pltpu.SemaphoreType.DMA,
            pltpu.SemaphoreType.DMA,
        ],
    )
    return f(user_ids.astype(jnp.int32), item_ids.astype(jnp.int32),
             user_table, item_table)


# speedup vs baseline: 1.0432x; 1.0123x over previous
import jax
import jax.numpy as jnp
from jax import lax
from jax.experimental import pallas as pl
from jax.experimental.pallas import tpu as pltpu
from jax.experimental.pallas import tpu_sc as plsc

BATCH = 16384
D = 128
NC = 2
NS = 16
NW = NC * NS
BPW = BATCH // NW
C = 128
NCHUNK = BPW // C
NBUF = 2
LANES = 16


def _gmf_body(uid_hbm, iid_hbm, ut_hbm, it_hbm, out_hbm,
              idx_u, idx_i, u_rows, i_rows, o_rows,
              sg0, sg1, so0, so1, sem_idx):
    sem_g = [sg0, sg1]
    sem_o = [so0, so1]
    wid = lax.axis_index("s") * NC + lax.axis_index("c")
    base = wid * BPW

    ci_u = pltpu.async_copy(uid_hbm.at[pl.ds(base, BPW)], idx_u, sem_idx)
    ci_i = pltpu.async_copy(iid_hbm.at[pl.ds(base, BPW)], idx_i, sem_idx)
    ci_u.wait()
    ci_i.wait()

    def issue_gather(k):
        b = k % NBUF
        sl = pl.ds(k * C, C)
        cu = pltpu.async_copy(ut_hbm.at[idx_u.at[sl]], u_rows.at[b], sem_g[b])
        ci = pltpu.async_copy(it_hbm.at[idx_i.at[sl]], i_rows.at[b], sem_g[b])
        return (cu, ci)

    pend_g = [None] * NCHUNK
    pend_o = [None] * NCHUNK
    pend_g[0] = issue_gather(0)
    pend_g[1] = issue_gather(1)
    for k in range(NCHUNK):
        b = k % NBUF
        cu, ci = pend_g[k]
        cu.wait()
        ci.wait()
        if k >= NBUF:
            pend_o[k - NBUF].wait()

        @plsc.parallel_loop(0, C, step=1, unroll=2)
        def _(r):
            for c in range(D // LANES):
                sl = pl.ds(c * LANES, LANES)
                o_rows[b, r, sl] = u_rows[b, r, sl] * i_rows[b, r, sl]

        if k + NBUF < NCHUNK:
            pend_g[k + NBUF] = issue_gather(k + NBUF)
        off = base + k * C
        pend_o[k] = pltpu.async_copy(
            o_rows.at[b], out_hbm.at[pl.ds(off, C)], sem_o[b])
    for k in range(max(0, NCHUNK - NBUF), NCHUNK):
        pend_o[k].wait()


def kernel(user_ids, item_ids, user_table, item_table):
    mesh = plsc.VectorSubcoreMesh(core_axis_name="c", subcore_axis_name="s")
    f = pl.kernel(
        _gmf_body,
        out_type=jax.ShapeDtypeStruct((BATCH, D), jnp.float32),
        mesh=mesh,
        scratch_types=[
            pltpu.VMEM((BPW,), jnp.int32),
            pltpu.VMEM((BPW,), jnp.int32),
            pltpu.VMEM((NBUF, C, D), jnp.float32),
            pltpu.VMEM((NBUF, C, D), jnp.float32),
            pltpu.VMEM((NBUF, C, D), jnp.float32),
            pltpu.SemaphoreType.DMA,
            pltpu.SemaphoreType.DMA,
            pltpu.SemaphoreType.DMA,
            pltpu.SemaphoreType.DMA,
            pltpu.SemaphoreType.DMA,
        ],
    )
    return f(user_ids.astype(jnp.int32), item_ids.astype(jnp.int32),
             user_table, item_table)


# speedup vs baseline: 1.0542x; 1.0106x over previous
import jax
import jax.numpy as jnp
from jax import lax
from jax.experimental import pallas as pl
from jax.experimental.pallas import tpu as pltpu
from jax.experimental.pallas import tpu_sc as plsc

BATCH = 16384
D = 128
NC = 2
NS = 16
NW = NC * NS
BPW = BATCH // NW
C = 128
NCHUNK = BPW // C
NBUF = 2
LANES = 16


def _gmf_body(uid_hbm, iid_hbm, ut_hbm, it_hbm, out_hbm,
              idx_u, idx_i, u_rows, i_rows, o_rows,
              sg0, sg1, so0, so1, sem_idx):
    sem_g = [sg0, sg1]
    sem_o = [so0, so1]
    wid = lax.axis_index("s") * NC + lax.axis_index("c")
    base = wid * BPW

    ci_u = pltpu.async_copy(uid_hbm.at[pl.ds(base, BPW)], idx_u, sem_idx)
    ci_i = pltpu.async_copy(iid_hbm.at[pl.ds(base, BPW)], idx_i, sem_idx)
    ci_u.wait()
    ci_i.wait()

    def issue_gather(k):
        b = k % NBUF
        sl = pl.ds(k * C, C)
        cu = pltpu.async_copy(ut_hbm.at[idx_u.at[sl]], u_rows.at[b], sem_g[b])
        ci = pltpu.async_copy(it_hbm.at[idx_i.at[sl]], i_rows.at[b], sem_g[b])
        return (cu, ci)

    pend_g = [None] * NCHUNK
    pend_o = [None] * NCHUNK
    pend_g[0] = issue_gather(0)
    pend_g[1] = issue_gather(1)
    for k in range(NCHUNK):
        b = k % NBUF
        cu, ci = pend_g[k]
        cu.wait()
        ci.wait()
        if k >= NBUF:
            pend_o[k - NBUF].wait()

        @plsc.parallel_loop(0, C, step=1, unroll=1)
        def _(r):
            for c in range(D // LANES):
                sl = pl.ds(c * LANES, LANES)
                o_rows[b, r, sl] = u_rows[b, r, sl] * i_rows[b, r, sl]

        if k + NBUF < NCHUNK:
            pend_g[k + NBUF] = issue_gather(k + NBUF)
        off = base + k * C
        pend_o[k] = pltpu.async_copy(
            o_rows.at[b], out_hbm.at[pl.ds(off, C)], sem_o[b])
    for k in range(max(0, NCHUNK - NBUF), NCHUNK):
        pend_o[k].wait()


def kernel(user_ids, item_ids, user_table, item_table):
    mesh = plsc.VectorSubcoreMesh(core_axis_name="c", subcore_axis_name="s")
    f = pl.kernel(
        _gmf_body,
        out_type=jax.ShapeDtypeStruct((BATCH, D), jnp.float32),
        mesh=mesh,
        scratch_types=[
            pltpu.VMEM((BPW,), jnp.int32),
            pltpu.VMEM((BPW,), jnp.int32),
            pltpu.VMEM((NBUF, C, D), jnp.float32),
            pltpu.VMEM((NBUF, C, D), jnp.float32),
            pltpu.VMEM((NBUF, C, D), jnp.float32),
            pltpu.SemaphoreType.DMA,
            pltpu.SemaphoreType.DMA,
            pltpu.SemaphoreType.DMA,
            pltpu.SemaphoreType.DMA,
            pltpu.SemaphoreType.DMA,
        ],
    )
    return f(user_ids.astype(jnp.int32), item_ids.astype(jnp.int32),
             user_table, item_table)


# speedup vs baseline: 1.0859x; 1.0300x over previous
import jax
import jax.numpy as jnp
from jax import lax
from jax.experimental import pallas as pl
from jax.experimental.pallas import tpu as pltpu
from jax.experimental.pallas import tpu_sc as plsc

BATCH = 16384
D = 128
NC = 2
NS = 16
NW = NC * NS
BPW = BATCH // NW
C = 64
NCHUNK = BPW // C
NBUF = 4
LANES = 16


def _gmf_body(uid_hbm, iid_hbm, ut_hbm, it_hbm, out_hbm,
              idx_u, idx_i, u_rows, i_rows, o_rows,
              sg0, sg1, sg2, sg3, so0, so1, so2, so3, sem_idx):
    sem_g = [sg0, sg1, sg2, sg3]
    sem_o = [so0, so1, so2, so3]
    wid = lax.axis_index("s") * NC + lax.axis_index("c")
    base = wid * BPW

    ci_u = pltpu.async_copy(uid_hbm.at[pl.ds(base, BPW)], idx_u, sem_idx)
    ci_i = pltpu.async_copy(iid_hbm.at[pl.ds(base, BPW)], idx_i, sem_idx)
    ci_u.wait()
    ci_i.wait()

    def issue_gather(k):
        b = k % NBUF
        sl = pl.ds(k * C, C)
        cu = pltpu.async_copy(ut_hbm.at[idx_u.at[sl]], u_rows.at[b], sem_g[b])
        ci = pltpu.async_copy(it_hbm.at[idx_i.at[sl]], i_rows.at[b], sem_g[b])
        return (cu, ci)

    pend_g = [None] * NCHUNK
    pend_o = [None] * NCHUNK
    for k in range(NBUF):
        pend_g[k] = issue_gather(k)
    for k in range(NCHUNK):
        b = k % NBUF
        cu, ci = pend_g[k]
        cu.wait()
        ci.wait()
        if k >= NBUF:
            pend_o[k - NBUF].wait()

        @plsc.parallel_loop(0, C, step=1, unroll=1)
        def _(r):
            for c in range(D // LANES):
                sl = pl.ds(c * LANES, LANES)
                o_rows[b, r, sl] = u_rows[b, r, sl] * i_rows[b, r, sl]

        if k + NBUF < NCHUNK:
            pend_g[k + NBUF] = issue_gather(k + NBUF)
        off = base + k * C
        pend_o[k] = pltpu.async_copy(
            o_rows.at[b], out_hbm.at[pl.ds(off, C)], sem_o[b])
    for k in range(max(0, NCHUNK - NBUF), NCHUNK):
        pend_o[k].wait()


def kernel(user_ids, item_ids, user_table, item_table):
    mesh = plsc.VectorSubcoreMesh(core_axis_name="c", subcore_axis_name="s")
    f = pl.kernel(
        _gmf_body,
        out_type=jax.ShapeDtypeStruct((BATCH, D), jnp.float32),
        mesh=mesh,
        scratch_types=[
            pltpu.VMEM((BPW,), jnp.int32),
            pltpu.VMEM((BPW,), jnp.int32),
            pltpu.VMEM((NBUF, C, D), jnp.float32),
            pltpu.VMEM((NBUF, C, D), jnp.float32),
            pltpu.VMEM((NBUF, C, D), jnp.float32),
            pltpu.SemaphoreType.DMA,
            pltpu.SemaphoreType.DMA,
            pltpu.SemaphoreType.DMA,
            pltpu.SemaphoreType.DMA,
            pltpu.SemaphoreType.DMA,
            pltpu.SemaphoreType.DMA,
            pltpu.SemaphoreType.DMA,
            pltpu.SemaphoreType.DMA,
            pltpu.SemaphoreType.DMA,
        ],
    )
    return f(user_ids.astype(jnp.int32), item_ids.astype(jnp.int32),
             user_table, item_table)
